# Initial kernel scaffold; baseline (speedup 1.0000x reference)
import jax
import jax.numpy as jnp
from jax.experimental import pallas as pl


def kernel(x, edge_index, W1, b1, W2, b2, Wih, Whh, bih, bhh, fc1W, fc1b, fc2W, fc2b):
    raise NotImplementedError("write your pallas kernel here")


# speedup vs baseline: 38.7689x; 38.7689x over previous
import functools

import jax
import jax.numpy as jnp
from jax import lax
from jax.experimental import pallas as pl
from jax.experimental.pallas import tpu as pltpu
from jax.experimental.pallas import tpu_sc as plsc

N_NODES = 10000
D = 128
NPAD = 10240
E = 320000
NW = 32
CHUNK = 128
KCH = 79
E_PAD = NW * KCH * CHUNK
TSTEPS = 10
BATCH = 1000
HID = 128
HID2 = 64
G4 = 4 * HID

_sc_mesh = plsc.VectorSubcoreMesh(core_axis_name="c", subcore_axis_name="s")


def _sc_deg_body(dst_hbm, deg_hbm, idx_v, ones_v, zero_v, hist_sh):
    cid = lax.axis_index("c")
    sid = lax.axis_index("s")
    wid = cid * 16 + sid

    ones16 = jnp.ones((16,), jnp.float32)
    zeros16 = jnp.zeros((16,), jnp.float32)
    for i in range(8):
        ones_v[pl.ds(i * 16, 16)] = ones16

    def _zfill(i, _):
        zero_v[pl.ds(i * 16, 16)] = zeros16
        return 0
    lax.fori_loop(0, 40, _zfill, 0)

    pltpu.sync_copy(zero_v, hist_sh.at[pl.ds(sid * 640, 640)])
    pltpu.sync_copy(dst_hbm.at[wid], idx_v)
    plsc.subcore_barrier()

    def _scat(j, _):
        pltpu.sync_copy(ones_v, hist_sh.at[idx_v.at[j]], add=True)
        return 0
    lax.fori_loop(0, KCH, _scat, 0)

    plsc.subcore_barrier()
    pltpu.sync_copy(hist_sh.at[pl.ds(sid * 640, 640)],
                    deg_hbm.at[cid, pl.ds(sid * 640, 640)])


_sc_deg = functools.partial(
    pl.kernel,
    out_type=jax.ShapeDtypeStruct((2, NPAD), jnp.float32),
    mesh=_sc_mesh,
    scratch_types=[
        pltpu.VMEM((KCH, CHUNK), jnp.int32),
        pltpu.VMEM((CHUNK,), jnp.float32),
        pltpu.VMEM((640,), jnp.float32),
        pltpu.VMEM_SHARED((NPAD,), jnp.float32),
    ],
)(_sc_deg_body)


def _sc_agg_body(y_hbm, src_hbm, dst_hbm, z_hbm,
                 sidx_v, didx_v, rows_v, zbuf_v, acc_sh, sem):
    cid = lax.axis_index("c")
    sid = lax.axis_index("s")
    wid = cid * 16 + sid

    zeros16 = jnp.zeros((16,), jnp.float32)

    def _zfill(i, _):
        for k in range(8):
            zbuf_v[i, pl.ds(k * 16, 16)] = zeros16
        return 0
    lax.fori_loop(0, 80, _zfill, 0)

    for t in range(8):
        pltpu.sync_copy(zbuf_v, acc_sh.at[pl.ds(sid * 640 + t * 80, 80)])
    pltpu.sync_copy(src_hbm.at[wid], sidx_v)
    pltpu.sync_copy(dst_hbm.at[wid], didx_v)
    plsc.subcore_barrier()

    def _edge(j, _):
        pltpu.async_copy(y_hbm.at[sidx_v.at[j]], rows_v, sem).wait()
        pltpu.sync_copy(rows_v, acc_sh.at[didx_v.at[j]], add=True)
        return 0
    lax.fori_loop(0, KCH, _edge, 0)

    plsc.subcore_barrier()
    pltpu.sync_copy(acc_sh.at[pl.ds(sid * 640, 640)],
                    z_hbm.at[cid, pl.ds(sid * 640, 640)])


_sc_agg = functools.partial(
    pl.kernel,
    out_type=jax.ShapeDtypeStruct((2, NPAD, D), jnp.float32),
    mesh=_sc_mesh,
    scratch_types=[
        pltpu.VMEM((KCH, CHUNK), jnp.int32),
        pltpu.VMEM((KCH, CHUNK), jnp.int32),
        pltpu.VMEM((CHUNK, D), jnp.float32),
        pltpu.VMEM((80, D), jnp.float32),
        pltpu.VMEM_SHARED((NPAD, D), jnp.float32),
        pltpu.SemaphoreType.DMA,
    ],
)(_sc_agg_body)


def _scale_body(x_ref, d_ref, o_ref):
    o_ref[...] = x_ref[...] * d_ref[...]


def _tc_scale(x_pad, dinv_col):
    return pl.pallas_call(
        _scale_body,
        grid=(8,),
        in_specs=[
            pl.BlockSpec((1280, D), lambda i: (i, 0)),
            pl.BlockSpec((1280, 1), lambda i: (i, 0)),
        ],
        out_specs=pl.BlockSpec((1280, D), lambda i: (i, 0)),
        out_shape=jax.ShapeDtypeStruct((NPAD, D), jnp.float32),
    )(x_pad, dinv_col)


def _gcn_body(zp_ref, y_ref, d_ref, w_ref, b_ref, o_ref):
    zsum = zp_ref[0] + zp_ref[1] + y_ref[...]
    agg = zsum * d_ref[...]
    g = jnp.dot(agg, w_ref[...], preferred_element_type=jnp.float32)
    g = g + b_ref[...]
    o_ref[...] = jax.nn.relu(g[:, :HID]) + jax.nn.relu(g[:, HID:])


def _tc_gcn(zp, y_pad, dinv_col, W12t, b12):
    return pl.pallas_call(
        _gcn_body,
        grid=(8,),
        in_specs=[
            pl.BlockSpec((2, 1280, D), lambda i: (0, i, 0)),
            pl.BlockSpec((1280, D), lambda i: (i, 0)),
            pl.BlockSpec((1280, 1), lambda i: (i, 0)),
            pl.BlockSpec((D, 2 * HID), lambda i: (0, 0)),
            pl.BlockSpec((1, 2 * HID), lambda i: (0, 0)),
        ],
        out_specs=pl.BlockSpec((1280, HID), lambda i: (i, 0)),
        out_shape=jax.ShapeDtypeStruct((NPAD, HID), jnp.float32),
    )(zp, y_pad, dinv_col, W12t, b12)


def _lstm_body(h_ref, wih_ref, whh_ref, bs_ref, f1w_ref, f1b_ref,
               f2w_ref, f2b_ref, o_ref, gx_ref):
    hv = h_ref[pl.ds(0, N_NODES), :]
    gx_ref[...] = jnp.dot(hv, wih_ref[...], preferred_element_type=jnp.float32)
    whh = whh_ref[...]
    bs = bs_ref[...]

    def _sigm(v):
        return 1.0 / (1.0 + jnp.exp(-v))

    def _step(t, carry):
        hp, cp = carry
        gates = gx_ref[pl.ds(t * BATCH, BATCH), :]
        gates = gates + jnp.dot(hp, whh, preferred_element_type=jnp.float32)
        gates = gates + bs
        ig = _sigm(gates[:, 0 * HID:1 * HID])
        fg = _sigm(gates[:, 1 * HID:2 * HID])
        gg = jnp.tanh(gates[:, 2 * HID:3 * HID])
        og = _sigm(gates[:, 3 * HID:4 * HID])
        c = fg * cp + ig * gg
        hn = og * jnp.tanh(c)
        return (hn, c)

    h0 = jnp.zeros((BATCH, HID), jnp.float32)
    c0 = jnp.zeros((BATCH, HID), jnp.float32)
    hT, _ = lax.fori_loop(0, TSTEPS, _step, (h0, c0))
    u = jax.nn.relu(jnp.dot(hT, f1w_ref[...], preferred_element_type=jnp.float32)
                    + f1b_ref[...])
    o_ref[...] = jnp.dot(u, f2w_ref[...],
                         preferred_element_type=jnp.float32) + f2b_ref[...]


def _tc_lstm(H, Wiht, Whht, bsum, f1w, f1b, f2w, f2b):
    return pl.pallas_call(
        _lstm_body,
        out_shape=jax.ShapeDtypeStruct((BATCH, HID), jnp.float32),
        scratch_shapes=[pltpu.VMEM((N_NODES, G4), jnp.float32)],
    )(H, Wiht, Whht, bsum, f1w, f1b, f2w, f2b)


def kernel(x, edge_index, W1, b1, W2, b2, Wih, Whh, bih, bhh,
           fc1W, fc1b, fc2W, fc2b):
    n_fill = E_PAD - E
    pad_idx = (N_NODES
               + jnp.arange(n_fill, dtype=jnp.int32) % (NPAD - N_NODES))
    src_p = jnp.concatenate([edge_index[0], pad_idx]).reshape(NW, KCH, CHUNK)
    dst_p = jnp.concatenate([edge_index[1], pad_idx]).reshape(NW, KCH, CHUNK)

    degp = _sc_deg(dst_p)
    deg = degp[0] + degp[1] + 1.0
    dinv_col = lax.rsqrt(deg)[:, None]

    x_pad = jnp.pad(x, ((0, NPAD - N_NODES), (0, 0)))
    y_pad = _tc_scale(x_pad, dinv_col)

    zp = _sc_agg(y_pad, src_p, dst_p)

    W12t = jnp.concatenate([W1.T, W2.T], axis=1)
    b12 = jnp.concatenate([b1, b2])[None]
    H = _tc_gcn(zp, y_pad, dinv_col, W12t, b12)

    return _tc_lstm(H, Wih.T, Whh.T, (bih + bhh)[None],
                    fc1W.T, fc1b[None], fc2W.T, fc2b[None])


# speedup vs baseline: 48.9629x; 1.2629x over previous
import functools

import jax
import jax.numpy as jnp
from jax import lax
from jax.experimental import pallas as pl
from jax.experimental.pallas import tpu as pltpu
from jax.experimental.pallas import tpu_sc as plsc

N_NODES = 10000
D = 128
NPAD = 10240
E = 320000
NW = 32
CHUNK = 128
KCH = 80
KBLK = 16
E_PAD = NW * KCH * CHUNK
TSTEPS = 10
BATCH = 1000
HID = 128
HID2 = 64
G4 = 4 * HID

_sc_mesh = plsc.VectorSubcoreMesh(core_axis_name="c", subcore_axis_name="s")


def _sc_deg_body(dst_hbm, deg_hbm, idx_v, ones_v, zero_v, hist_sh):
    cid = lax.axis_index("c")
    sid = lax.axis_index("s")
    wid = cid * 16 + sid

    ones16 = jnp.ones((16,), jnp.float32)
    zeros16 = jnp.zeros((16,), jnp.float32)
    for i in range(8):
        ones_v[pl.ds(i * 16, 16)] = ones16

    def _zfill(i, _):
        zero_v[pl.ds(i * 16, 16)] = zeros16
        return 0
    lax.fori_loop(0, 40, _zfill, 0)

    pltpu.sync_copy(zero_v, hist_sh.at[pl.ds(sid * 640, 640)])
    pltpu.sync_copy(dst_hbm.at[wid], idx_v)
    plsc.subcore_barrier()

    def _scat(j, _):
        pltpu.sync_copy(ones_v, hist_sh.at[idx_v.at[j]], add=True)
        return 0
    lax.fori_loop(0, KCH, _scat, 0)

    plsc.subcore_barrier()
    pltpu.sync_copy(hist_sh.at[pl.ds(sid * 640, 640)],
                    deg_hbm.at[cid, pl.ds(sid * 640, 640)])


_sc_deg = functools.partial(
    pl.kernel,
    out_type=jax.ShapeDtypeStruct((2, NPAD), jnp.float32),
    mesh=_sc_mesh,
    scratch_types=[
        pltpu.VMEM((KCH, CHUNK), jnp.int32),
        pltpu.VMEM((CHUNK,), jnp.float32),
        pltpu.VMEM((640,), jnp.float32),
        pltpu.VMEM_SHARED((NPAD,), jnp.float32),
    ],
)(_sc_deg_body)


def _sc_agg_body(y_hbm, src_hbm, dst_hbm, z_hbm,
                 sidx_v, didx_v, rows0_v, rows1_v, zbuf_v, acc_sh,
                 sem0, sem1):
    cid = lax.axis_index("c")
    sid = lax.axis_index("s")
    wid = cid * 16 + sid

    zeros16 = jnp.zeros((16,), jnp.float32)

    def _zfill(i, _):
        for k in range(8):
            zbuf_v[i, pl.ds(k * 16, 16)] = zeros16
        return 0
    lax.fori_loop(0, KBLK, _zfill, 0)

    for t in range(40):
        pltpu.sync_copy(zbuf_v, acc_sh.at[pl.ds(sid * 640 + t * 16, 16)])
    plsc.subcore_barrier()

    def _gather(j, rows, sem):
        pltpu.async_copy(y_hbm.at[sidx_v.at[j]], rows, sem)

    def _gwait(j, rows, sem):
        pltpu.make_async_copy(y_hbm.at[sidx_v.at[j]], rows, sem).wait()

    def _scat(j, rows):
        pltpu.sync_copy(rows, acc_sh.at[didx_v.at[j]], add=True)

    for b in range(KCH // KBLK):
        pltpu.sync_copy(src_hbm.at[wid, pl.ds(b * KBLK, KBLK)], sidx_v)
        pltpu.sync_copy(dst_hbm.at[wid, pl.ds(b * KBLK, KBLK)], didx_v)
        _gather(0, rows0_v, sem0)

        def _edge2(i, _):
            j0 = 2 * i
            _gather(j0 + 1, rows1_v, sem1)
            _gwait(j0, rows0_v, sem0)
            _scat(j0, rows0_v)
            _gather(j0 + 2, rows0_v, sem0)
            _gwait(j0 + 1, rows1_v, sem1)
            _scat(j0 + 1, rows1_v)
            return 0
        lax.fori_loop(0, KBLK // 2 - 1, _edge2, 0)

        _gather(KBLK - 1, rows1_v, sem1)
        _gwait(KBLK - 2, rows0_v, sem0)
        _scat(KBLK - 2, rows0_v)
        _gwait(KBLK - 1, rows1_v, sem1)
        _scat(KBLK - 1, rows1_v)

    plsc.subcore_barrier()
    pltpu.sync_copy(acc_sh.at[pl.ds(sid * 640, 640)],
                    z_hbm.at[cid, pl.ds(sid * 640, 640)])


_sc_agg = functools.partial(
    pl.kernel,
    out_type=jax.ShapeDtypeStruct((2, NPAD, D), jnp.float32),
    mesh=_sc_mesh,
    scratch_types=[
        pltpu.VMEM((KBLK, CHUNK), jnp.int32),
        pltpu.VMEM((KBLK, CHUNK), jnp.int32),
        pltpu.VMEM((CHUNK, D), jnp.float32),
        pltpu.VMEM((CHUNK, D), jnp.float32),
        pltpu.VMEM((KBLK, D), jnp.float32),
        pltpu.VMEM_SHARED((NPAD, D), jnp.float32),
        pltpu.SemaphoreType.DMA,
        pltpu.SemaphoreType.DMA,
    ],
)(_sc_agg_body)


def _scale_body(x_ref, d_ref, o_ref):
    o_ref[...] = x_ref[...] * d_ref[...]


def _tc_scale(x_pad, dinv_col):
    return pl.pallas_call(
        _scale_body,
        grid=(8,),
        in_specs=[
            pl.BlockSpec((1280, D), lambda i: (i, 0)),
            pl.BlockSpec((1280, 1), lambda i: (i, 0)),
        ],
        out_specs=pl.BlockSpec((1280, D), lambda i: (i, 0)),
        out_shape=jax.ShapeDtypeStruct((NPAD, D), jnp.float32),
    )(x_pad, dinv_col)


def _gcn_body(zp_ref, y_ref, d_ref, w_ref, b_ref, o_ref):
    zsum = zp_ref[0] + zp_ref[1] + y_ref[...]
    agg = zsum * d_ref[...]
    g = jnp.dot(agg, w_ref[...], preferred_element_type=jnp.float32)
    g = g + b_ref[...]
    o_ref[...] = jax.nn.relu(g[:, :HID]) + jax.nn.relu(g[:, HID:])


def _tc_gcn(zp, y_pad, dinv_col, W12t, b12):
    return pl.pallas_call(
        _gcn_body,
        grid=(8,),
        in_specs=[
            pl.BlockSpec((2, 1280, D), lambda i: (0, i, 0)),
            pl.BlockSpec((1280, D), lambda i: (i, 0)),
            pl.BlockSpec((1280, 1), lambda i: (i, 0)),
            pl.BlockSpec((D, 2 * HID), lambda i: (0, 0)),
            pl.BlockSpec((1, 2 * HID), lambda i: (0, 0)),
        ],
        out_specs=pl.BlockSpec((1280, HID), lambda i: (i, 0)),
        out_shape=jax.ShapeDtypeStruct((NPAD, HID), jnp.float32),
    )(zp, y_pad, dinv_col, W12t, b12)


def _lstm_body(h_ref, wih_ref, whh_ref, bs_ref, f1w_ref, f1b_ref,
               f2w_ref, f2b_ref, o_ref, gx_ref):
    hv = h_ref[pl.ds(0, N_NODES), :]
    gx_ref[...] = jnp.dot(hv, wih_ref[...], preferred_element_type=jnp.float32)
    whh = whh_ref[...]
    bs = bs_ref[...]

    def _sigm(v):
        return 1.0 / (1.0 + jnp.exp(-v))

    def _step(t, carry):
        hp, cp = carry
        gates = gx_ref[pl.ds(t * BATCH, BATCH), :]
        gates = gates + jnp.dot(hp, whh, preferred_element_type=jnp.float32)
        gates = gates + bs
        ig = _sigm(gates[:, 0 * HID:1 * HID])
        fg = _sigm(gates[:, 1 * HID:2 * HID])
        gg = jnp.tanh(gates[:, 2 * HID:3 * HID])
        og = _sigm(gates[:, 3 * HID:4 * HID])
        c = fg * cp + ig * gg
        hn = og * jnp.tanh(c)
        return (hn, c)

    h0 = jnp.zeros((BATCH, HID), jnp.float32)
    c0 = jnp.zeros((BATCH, HID), jnp.float32)
    hT, _ = lax.fori_loop(0, TSTEPS, _step, (h0, c0))
    u = jax.nn.relu(jnp.dot(hT, f1w_ref[...], preferred_element_type=jnp.float32)
                    + f1b_ref[...])
    o_ref[...] = jnp.dot(u, f2w_ref[...],
                         preferred_element_type=jnp.float32) + f2b_ref[...]


def _tc_lstm(H, Wiht, Whht, bsum, f1w, f1b, f2w, f2b):
    return pl.pallas_call(
        _lstm_body,
        out_shape=jax.ShapeDtypeStruct((BATCH, HID), jnp.float32),
        scratch_shapes=[pltpu.VMEM((N_NODES, G4), jnp.float32)],
    )(H, Wiht, Whht, bsum, f1w, f1b, f2w, f2b)


def kernel(x, edge_index, W1, b1, W2, b2, Wih, Whh, bih, bhh,
           fc1W, fc1b, fc2W, fc2b):
    n_fill = E_PAD - E
    pad_idx = (N_NODES
               + jnp.arange(n_fill, dtype=jnp.int32) % (NPAD - N_NODES))
    src_p = jnp.concatenate([edge_index[0], pad_idx]).reshape(NW, KCH, CHUNK)
    dst_p = jnp.concatenate([edge_index[1], pad_idx]).reshape(NW, KCH, CHUNK)

    degp = _sc_deg(dst_p)
    deg = degp[0] + degp[1] + 1.0
    dinv_col = lax.rsqrt(deg)[:, None]

    x_pad = jnp.pad(x, ((0, NPAD - N_NODES), (0, 0)))
    y_pad = _tc_scale(x_pad, dinv_col)

    zp = _sc_agg(y_pad, src_p, dst_p)

    W12t = jnp.concatenate([W1.T, W2.T], axis=1)
    b12 = jnp.concatenate([b1, b2])[None]
    H = _tc_gcn(zp, y_pad, dinv_col, W12t, b12)

    return _tc_lstm(H, Wih.T, Whh.T, (bih + bhh)[None],
                    fc1W.T, fc1b[None], fc2W.T, fc2b[None])
